# baseline (speedup 1.0000x reference)
import functools

import jax
import jax.numpy as jnp
import numpy as np
from jax import lax
from jax.experimental import pallas as pl
from jax.experimental.pallas import tpu as pltpu
from jax.experimental.pallas import tpu_sc as plsc

C, S, W, K = 64, 8, 1024, 32
NC, NS, L = 2, 16, 16
NW = NC * NS
CPW = C // NW
ELEMS_PW = CPW * S * W
WORDS_PC = W * (K // 8)
WORDS_PW = CPW * WORDS_PC


@functools.cache
def _get_sc_hist():
    mesh = plsc.VectorSubcoreMesh(core_axis_name="c", subcore_axis_name="s")

    @functools.partial(
        pl.kernel,
        mesh=mesh,
        compiler_params=pltpu.CompilerParams(needs_layout_passes=False),
        out_type=(
            jax.ShapeDtypeStruct((NW * WORDS_PW,), jnp.int32),
            jax.ShapeDtypeStruct((NW * L,), jnp.float32),
        ),
        scratch_types=[
            pltpu.VMEM((ELEMS_PW,), jnp.int32),
            pltpu.VMEM((WORDS_PW,), jnp.int32),
            pltpu.VMEM((9 * L,), jnp.float32),
            pltpu.SemaphoreType.DMA,
            pltpu.SemaphoreType.DMA,
        ],
    )
    def _sc_hist(t_hbm, g_hbm, out_hbm, insum_hbm, t_v, cnt_v, g_v, in_sem, g_sem):
        wid = lax.axis_index("s") * NC + lax.axis_index("c")
        in_cp = pltpu.async_copy(
            t_hbm.at[pl.ds(wid * ELEMS_PW, ELEMS_PW)], t_v, in_sem
        )
        g_cp = pltpu.async_copy(g_hbm, g_v, g_sem)

        zeros = jnp.zeros((L,), jnp.int32)

        @plsc.parallel_loop(0, WORDS_PW // L, 1, unroll=4)
        def zbody(i):
            cnt_v[pl.ds(i * L, L)] = zeros

        in_cp.wait()

        lanes4 = lax.iota(jnp.int32, L) * (K // 8)
        n_chunks = S * W // L

        @plsc.parallel_loop(0, n_chunks, 1, unroll=4)
        def body(i):
            base = ((i >> 6) << 9) + ((i & 7) << 6)
            for c_local in range(CPW):
                t = t_v[pl.ds(c_local * S * W + i * L, L)]
                tm1 = t - 1
                val = jnp.int32(1) << ((tm1 & 7) << 2)
                idx = (tm1 >> 3) + (lanes4 + (base + c_local * WORDS_PC))
                plsc.addupdate_scatter(cnt_v, [idx], val)
        out_cp = pltpu.async_copy(
            cnt_v, out_hbm.at[pl.ds(wid * WORDS_PW, WORDS_PW)], in_sem
        )

        g_cp.wait()
        lanes1 = lax.iota(jnp.int32, L)

        @plsc.parallel_loop(0, n_chunks, 1, unroll=4,
                            carry=jnp.zeros((L,), jnp.float32))
        def acc_loop(i, acc):
            base = ((i >> 6) << 9) + ((i & 7) << 6)
            for c_local in range(CPW):
                t = t_v[pl.ds(c_local * S * W + i * L, L)]
                tm1 = t - 1
                idx = (tm1 >> 3) + (lanes4 + (base + c_local * WORDS_PC))
                word = plsc.load_gather(cnt_v, [idx])
                m = (word >> ((tm1 & 7) << 2)) & 15
                acc = acc + plsc.load_gather(g_v, [(m << 4) + lanes1])
            return acc

        g_v[pl.ds(0, L)] = acc_loop
        out2_cp = pltpu.async_copy(
            g_v.at[pl.ds(0, L)], insum_hbm.at[pl.ds(wid * L, L)], g_sem
        )
        out_cp.wait()
        out2_cp.wait()

    return _sc_hist


def _tc_reduce_body(pk_ref, part_ref, out_ref):
    p = pk_ref[...]
    inter_sum = jnp.float32(0.0)
    for j in range(8):
        n = ((p >> (4 * j)) & 15).astype(jnp.float32)
        s1 = jnp.sum(n, axis=0)
        s2 = jnp.sum(n * n, axis=0)
        d = jnp.maximum(s2 - s1 * s1 * (1.0 / C), 0.0)
        inter_sum += jnp.sum(jnp.sqrt(d))
    in_sum = jnp.sum(part_ref[...])
    scale_in = 1.0 / (C * (S * (S - 1)) ** 0.5)
    scale_inter = (S * S * (C - 1)) ** 0.5
    out_ref[0, 0] = scale_inter / inter_sum + in_sum * scale_in


def kernel(target):
    t_flat = target.reshape(C, S, W // 128, 128).transpose(0, 2, 1, 3).reshape(-1)
    gm = np.sqrt(np.arange(9) * (8.0 - np.arange(9)))
    gm[1:] /= np.arange(1, 9)
    gtab = jnp.asarray(np.repeat(gm, L).astype(np.float32))
    packed, insums = _get_sc_hist()(t_flat, gtab)
    packed = packed.reshape(C, WORDS_PC // 128, 128)
    res = pl.pallas_call(
        _tc_reduce_body,
        out_shape=jax.ShapeDtypeStruct((1, 1), jnp.float32),
        out_specs=pl.BlockSpec(memory_space=pltpu.SMEM),
    )(packed, insums.reshape(NW * L // 128, 128))
    return res[0, 0]


# speedup vs baseline: 1.0077x; 1.0077x over previous
import functools

import jax
import jax.numpy as jnp
import numpy as np
from jax import lax
from jax.experimental import pallas as pl
from jax.experimental.pallas import tpu as pltpu
from jax.experimental.pallas import tpu_sc as plsc

C, S, W, K = 64, 8, 1024, 32
NC, NS, L = 2, 16, 16
NW = NC * NS
CPW = C // NW
ELEMS_PW = CPW * S * W
WORDS_PC = W * (K // 8)
WORDS_PW = CPW * WORDS_PC


@functools.cache
def _get_sc_hist():
    mesh = plsc.VectorSubcoreMesh(core_axis_name="c", subcore_axis_name="s")

    @functools.partial(
        pl.kernel,
        mesh=mesh,
        compiler_params=pltpu.CompilerParams(needs_layout_passes=False),
        out_type=(
            jax.ShapeDtypeStruct((NW * WORDS_PW,), jnp.int32),
            jax.ShapeDtypeStruct((2 * NW * L,), jnp.float32),
        ),
        scratch_types=[
            pltpu.VMEM((ELEMS_PW,), jnp.int32),
            pltpu.VMEM((WORDS_PW,), jnp.int32),
            pltpu.VMEM((9 * L,), jnp.float32),
            pltpu.SemaphoreType.DMA,
            pltpu.SemaphoreType.DMA,
        ],
    )
    def _sc_hist(t_hbm, out_hbm, insum_hbm, t_v, cnt_v, g_v, in_sem, g_sem):
        wid = lax.axis_index("s") * NC + lax.axis_index("c")
        in_cp = pltpu.async_copy(
            t_hbm.at[pl.ds(wid * ELEMS_PW, ELEMS_PW)], t_v, in_sem
        )
        for m in range(1, 9):
            gm = float(np.sqrt(m * (8.0 - m)) / m)
            g_v[pl.ds(m * L, L)] = jnp.full((L,), gm, jnp.float32)

        zeros = jnp.zeros((L,), jnp.int32)

        @plsc.parallel_loop(0, WORDS_PW // L, 1, unroll=4)
        def zbody(i):
            cnt_v[pl.ds(i * L, L)] = zeros

        in_cp.wait()

        lanes4 = lax.iota(jnp.int32, L) * (K // 8)
        n_chunks = S * W // L

        @plsc.parallel_loop(0, n_chunks, 1, unroll=8)
        def body(i):
            base = ((i >> 6) << 9) + ((i & 7) << 6)
            for c_local in range(CPW):
                t = t_v[pl.ds(c_local * S * W + i * L, L)]
                tm1 = t - 1
                val = jnp.int32(1) << ((tm1 & 7) << 2)
                idx = (tm1 >> 3) + (lanes4 + (base + c_local * WORDS_PC))
                plsc.addupdate_scatter(cnt_v, [idx], val)
        out_cp = pltpu.async_copy(
            cnt_v, out_hbm.at[pl.ds(wid * WORDS_PW, WORDS_PW)], in_sem
        )

        lanes1 = lax.iota(jnp.int32, L)

        @plsc.parallel_loop(0, n_chunks, 1, unroll=4,
                            carry=jnp.zeros((L,), jnp.float32))
        def acc_loop(i, acc):
            base = ((i >> 6) << 9) + ((i & 7) << 6)
            for c_local in range(CPW):
                t = t_v[pl.ds(c_local * S * W + i * L, L)]
                tm1 = t - 1
                idx = (tm1 >> 3) + (lanes4 + (base + c_local * WORDS_PC))
                word = plsc.load_gather(cnt_v, [idx])
                m = (word >> ((tm1 & 7) << 2)) & 15
                acc = acc + plsc.load_gather(g_v, [(m << 4) + lanes1])
            return acc

        g_v[pl.ds(0, L)] = acc_loop
        out2_cp = pltpu.async_copy(
            g_v.at[pl.ds(0, L)], insum_hbm.at[pl.ds(wid * L, L)], g_sem
        )
        out2_cp.wait()
        g_v[pl.ds(0, L)] = jnp.zeros((L,), jnp.float32)
        out3_cp = pltpu.async_copy(
            g_v.at[pl.ds(0, L)], insum_hbm.at[pl.ds((NW + wid) * L, L)], g_sem
        )
        out_cp.wait()
        out3_cp.wait()

    return _sc_hist


def _tc_reduce_body(pk_ref, part_ref, out_ref):
    p = pk_ref[...]
    inter_sum = jnp.float32(0.0)
    for j in range(8):
        n = ((p >> (4 * j)) & 15).astype(jnp.float32)
        s1 = jnp.sum(n, axis=0)
        s2 = jnp.sum(n * n, axis=0)
        d = jnp.maximum(s2 - s1 * s1 * (1.0 / C), 0.0)
        inter_sum += jnp.sum(jnp.sqrt(d))
    in_sum = jnp.sum(part_ref[...])
    scale_in = 1.0 / (C * (S * (S - 1)) ** 0.5)
    scale_inter = (S * S * (C - 1)) ** 0.5
    out_ref[0, 0] = scale_inter / inter_sum + in_sum * scale_in


def kernel(target):
    t_flat = target.reshape(C, S, W // 128, 128).transpose(0, 2, 1, 3).reshape(-1)
    packed, insums = _get_sc_hist()(t_flat)
    packed = packed.reshape(C, WORDS_PC // 128, 128)
    res = pl.pallas_call(
        _tc_reduce_body,
        out_shape=jax.ShapeDtypeStruct((1, 1), jnp.float32),
        out_specs=pl.BlockSpec(memory_space=pltpu.SMEM),
    )(packed, insums.reshape(2 * NW * L // 128, 128))
    return res[0, 0]
